# baseline (speedup 1.0000x reference)
import functools

import jax
import jax.numpy as jnp
from jax import lax
from jax.experimental import pallas as pl
from jax.experimental.pallas import tpu as pltpu
from jax.experimental.pallas import tpu_sc as plsc

D = 64
B_ROWS = 4096
SEQ = 200
B = B_ROWS * SEQ
NC = 2
NS = 16
NW = NC * NS
CHUNK = 128
CPW = B // (NW * CHUNK)

_mesh = plsc.VectorSubcoreMesh(core_axis_name="c", subcore_axis_name="s")


@functools.partial(
    pl.kernel,
    out_type=jax.ShapeDtypeStruct((B, D), jnp.float32),
    mesh=_mesh,
    scratch_types=[
        pltpu.VMEM((CPW, CHUNK), jnp.int32),
        pltpu.VMEM((CHUNK, D), jnp.float32),
        pltpu.SemaphoreType.DMA,
    ],
    compiler_params=pltpu.CompilerParams(use_tc_tiling_on_sc=False),
)
def _gather_kernel(idx_hbm, table_hbm, out_hbm, idx_v, rows_v, sem):
    wid = lax.axis_index("s") * NC + lax.axis_index("c")
    pltpu.sync_copy(idx_hbm.at[wid], idx_v)
    base = wid * (CPW * CHUNK)

    def body(j, carry):
        pltpu.async_copy(table_hbm.at[idx_v.at[j]], rows_v, sem).wait()
        pltpu.sync_copy(rows_v, out_hbm.at[pl.ds(base + j * CHUNK, CHUNK)])
        return carry

    lax.fori_loop(0, CPW, body, 0)


def kernel(input_, weight):
    idx = input_.reshape(NW, CPW, CHUNK).astype(jnp.int32)
    out = _gather_kernel(idx, weight)
    return out.reshape(B_ROWS, SEQ, D)


# speedup vs baseline: 1.1154x; 1.1154x over previous
import functools

import jax
import jax.numpy as jnp
from jax import lax
from jax.experimental import pallas as pl
from jax.experimental.pallas import tpu as pltpu
from jax.experimental.pallas import tpu_sc as plsc

D = 64
B_ROWS = 4096
SEQ = 200
B = B_ROWS * SEQ
NC = 2
NS = 16
NW = NC * NS
CHUNK = 128
CPW = B // (NW * CHUNK)
NB = 8
LA = 4

_mesh = plsc.VectorSubcoreMesh(core_axis_name="c", subcore_axis_name="s")


@functools.partial(
    pl.kernel,
    out_type=jax.ShapeDtypeStruct((B, D), jnp.float32),
    mesh=_mesh,
    scratch_types=[
        pltpu.VMEM((CPW, CHUNK), jnp.int32),
        [pltpu.VMEM((CHUNK, D), jnp.float32)] * NB,
        [pltpu.SemaphoreType.DMA] * NB,
        [pltpu.SemaphoreType.DMA] * NB,
    ],
    compiler_params=pltpu.CompilerParams(use_tc_tiling_on_sc=False),
)
def _gather_kernel(idx_hbm, table_hbm, out_hbm, idx_v, rows, sem_g, sem_o):
    wid = lax.axis_index("s") * NC + lax.axis_index("c")
    pltpu.sync_copy(idx_hbm.at[wid], idx_v)
    base = wid * (CPW * CHUNK)

    def fire_gather(j, b):
        pltpu.async_copy(table_hbm.at[idx_v.at[j]], rows[b], sem_g[b])

    def wait_gather(j, b):
        pltpu.make_async_copy(table_hbm.at[idx_v.at[j]], rows[b],
                              sem_g[b]).wait()

    def out_ref(j):
        return out_hbm.at[pl.ds(base + j * CHUNK, CHUNK)]

    def fire_out(j, b):
        pltpu.async_copy(rows[b], out_ref(j), sem_o[b])

    def wait_out(j, b):
        pltpu.make_async_copy(rows[b], out_ref(j), sem_o[b]).wait()

    for b in range(LA):
        fire_gather(b, b)

    for b in range(NB):
        j = b
        wait_gather(j, b)
        fire_out(j, b)
        jn = j + LA
        bn = jn % NB
        if j >= LA:
            wait_out(j - LA, bn)
        fire_gather(jn, bn)

    def round_body(r, carry):
        j0 = r * NB
        for b in range(NB):
            j = j0 + b
            wait_gather(j, b)
            fire_out(j, b)
            jn = j + LA
            bn = (b + LA) % NB
            wait_out(j - LA, bn)
            fire_gather(jn, bn)
        return carry

    lax.fori_loop(1, CPW // NB - 1, round_body, 0)

    j0 = CPW - NB
    for b in range(NB):
        j = j0 + b
        wait_gather(j, b)
        fire_out(j, b)
        if b < LA:
            jn = j + LA
            bn = (b + LA) % NB
            wait_out(j - LA, bn)
            fire_gather(jn, bn)

    for b in range(NB):
        wait_out(j0 + b, b)


def kernel(input_, weight):
    idx = input_.reshape(NW, CPW, CHUNK).astype(jnp.int32)
    out = _gather_kernel(idx, weight)
    return out.reshape(B_ROWS, SEQ, D)
